# baseline (speedup 1.0000x reference)
import functools

import jax
import jax.numpy as jnp
from jax import lax
from jax.experimental import pallas as pl
from jax.experimental.pallas import tpu as pltpu
from jax.experimental.pallas import tpu_sc as plsc

N = 10000
E = 320000
IN_DIM, H1, H2 = 128, 64, 32

NC, NS, LANES = 2, 16, 16
NW = NC * NS
NPAD = 10240
EPW = E // NW
CH = 128
NFULL = EPW // CH
TAIL = EPW - NFULL * CH

@functools.lru_cache(maxsize=None)
def _sc_mesh():
    return plsc.VectorSubcoreMesh(core_axis_name="c", subcore_axis_name="s",
                                  num_cores=NC, num_subcores=NS)


def _deg_body(src_hbm, dst_hbm, out_hbm, sidx_v, didx_v, hs_v, hd_v,
              isem, isem2):
    c = lax.axis_index("c")
    s = lax.axis_index("s")
    wid = s * NC + c
    zeros = jnp.zeros((LANES,), jnp.float32)
    ones = jnp.ones((LANES,), jnp.float32)
    sdesc = pltpu.async_copy(src_hbm.at[pl.ds(wid * EPW, EPW)], sidx_v, isem)
    ddesc = pltpu.async_copy(dst_hbm.at[pl.ds(wid * EPW, EPW)], didx_v, isem2)

    def zero_body(i):
        hs_v[pl.ds(i * LANES, LANES)] = zeros
        hd_v[pl.ds(i * LANES, LANES)] = zeros
    pl.loop(0, NPAD // LANES, unroll=8)(zero_body)

    sdesc.wait()

    def scat_s(i):
        idx = sidx_v[pl.ds(i * LANES, LANES)]
        plsc.addupdate_scatter(hs_v, [idx], ones)
    pl.loop(0, EPW // LANES, unroll=8)(scat_s)
    pltpu.sync_copy(hs_v, out_hbm.at[0, wid])

    ddesc.wait()

    def scat_d(i):
        idx = didx_v[pl.ds(i * LANES, LANES)]
        plsc.addupdate_scatter(hd_v, [idx], ones)
    pl.loop(0, EPW // LANES, unroll=8)(scat_d)
    pltpu.sync_copy(hd_v, out_hbm.at[1, wid])


@functools.lru_cache(maxsize=None)
def _sc_degrees_kernel():
    return pl.kernel(
        _deg_body,
        out_type=jax.ShapeDtypeStruct((2, NW, NPAD), jnp.float32),
        mesh=_sc_mesh(),
        compiler_params=pltpu.CompilerParams(needs_layout_passes=False),
        scratch_types=[
            pltpu.VMEM((EPW,), jnp.int32),
            pltpu.VMEM((EPW,), jnp.int32),
            pltpu.VMEM((NPAD,), jnp.float32),
            pltpu.VMEM((NPAD,), jnp.float32),
            pltpu.SemaphoreType.DMA,
            pltpu.SemaphoreType.DMA,
        ],
    )


def _sc_degrees(src, dst):
    return _sc_degrees_kernel()(src, dst)


ROWS_PER_SUB = NPAD // NS


ACH = 200
ANCH = EPW // ACH
DEPTH = 4
ZB_ROWS = 128


def _agg_body(msg_hbm, src_hbm, dst_hbm, out_hbm,
              sidx_v, didx_v, rows0, rows1, rows2, rows3, zb_v, acc_sh,
              gsem, ssem):
    c = lax.axis_index("c")
    s = lax.axis_index("s")
    wid = s * NC + c
    zeros = jnp.zeros((LANES,), jnp.float32)
    rows = (rows0, rows1, rows2, rows3)

    pltpu.sync_copy(src_hbm.at[pl.ds(wid * EPW, EPW)], sidx_v)
    pltpu.sync_copy(dst_hbm.at[pl.ds(wid * EPW, EPW)], didx_v)

    def issue(k):
        return pltpu.async_copy(msg_hbm.at[sidx_v.at[pl.ds(k * ACH, ACH)]],
                                rows[k % DEPTH], gsem)

    gdescs = [None] * DEPTH
    sdescs = [None] * DEPTH
    for k in range(DEPTH - 1):
        gdescs[k] = issue(k)

    def zero_body(i):
        for j in range(H1 // LANES):
            zb_v[i, pl.ds(j * LANES, LANES)] = zeros
    pl.loop(0, ZB_ROWS)(zero_body)
    for q in range(ROWS_PER_SUB // ZB_ROWS):
        pltpu.sync_copy(zb_v, acc_sh.at[pl.ds(s * ROWS_PER_SUB + q * ZB_ROWS,
                                              ZB_ROWS)])
    plsc.subcore_barrier()

    for k in range(ANCH):
        b = k % DEPTH
        ka = k + DEPTH - 1
        if ka < ANCH:
            ba = ka % DEPTH
            if sdescs[ba] is not None:
                sdescs[ba].wait()
            gdescs[ba] = issue(ka)
        gdescs[b].wait()
        sdescs[b] = pltpu.async_copy(
            rows[b], acc_sh.at[didx_v.at[pl.ds(k * ACH, ACH)]], ssem,
            add=True)
    for k in range(max(0, ANCH - DEPTH), ANCH):
        sdescs[k % DEPTH].wait()
    plsc.subcore_barrier()
    pltpu.sync_copy(acc_sh.at[pl.ds(s * ROWS_PER_SUB, ROWS_PER_SUB)],
                    out_hbm.at[c, pl.ds(s * ROWS_PER_SUB, ROWS_PER_SUB)])


@functools.lru_cache(maxsize=None)
def _sc_agg_kernel():
    return pl.kernel(
        _agg_body,
        out_type=jax.ShapeDtypeStruct((NC, NPAD, H1), jnp.float32),
        mesh=_sc_mesh(),
        compiler_params=pltpu.CompilerParams(needs_layout_passes=False,
                                             use_tc_tiling_on_sc=False),
        scratch_types=[
            pltpu.VMEM((EPW,), jnp.int32),
            pltpu.VMEM((EPW,), jnp.int32),
            pltpu.VMEM((ACH, H1), jnp.float32),
            pltpu.VMEM((ACH, H1), jnp.float32),
            pltpu.VMEM((ACH, H1), jnp.float32),
            pltpu.VMEM((ACH, H1), jnp.float32),
            pltpu.VMEM((ZB_ROWS, H1), jnp.float32),
            pltpu.VMEM_SHARED((NPAD, H1), jnp.float32),
            pltpu.SemaphoreType.DMA,
            pltpu.SemaphoreType.DMA,
        ],
    )


def _sc_agg(msg, src, dst):
    return _sc_agg_kernel()(msg, src, dst)


def _tc_a_body(deg_ref, x_ref, w1_ref, h1pre_ref, rout_ref, rin_ref):
    deg = jnp.sum(deg_ref[...], axis=1)
    r = lax.rsqrt(jnp.maximum(deg, 1.0))
    r_out = jnp.reshape(r[0, :N], (N, 1))
    r_in = jnp.reshape(r[1, :N], (N, 1))
    rout_ref[...] = r_out
    rin_ref[...] = r_in
    h1pre_ref[...] = jnp.dot(x_ref[...] * r_out, w1_ref[...],
                             preferred_element_type=jnp.float32)


def _tc_a(deg_parts, features, W1):
    return pl.pallas_call(
        _tc_a_body,
        out_shape=(
            jax.ShapeDtypeStruct((N, H1), jnp.float32),
            jax.ShapeDtypeStruct((N, 1), jnp.float32),
            jax.ShapeDtypeStruct((N, 1), jnp.float32),
        ),
    )(deg_parts, features, W1)


def _tc_b_body(p_ref, rin_ref, rout_ref, b1_ref, w23_ref, m_ref):
    agg1 = p_ref[0, :N, :] + p_ref[1, :N, :]
    h = jnp.maximum(agg1 * rin_ref[...] + b1_ref[...], 0.0)
    m_ref[...] = jnp.dot(h * rout_ref[...], w23_ref[...],
                         preferred_element_type=jnp.float32)


def _tc_b(agg1_parts, r_in, r_out, b1, W23):
    return pl.pallas_call(
        _tc_b_body,
        out_shape=jax.ShapeDtypeStruct((N, H1), jnp.float32),
    )(agg1_parts, r_in, r_out, b1, W23)


def _tc_c_body(p_ref, rin_ref, b2_ref, b3_ref, noise_ref, z_ref):
    agg2 = (p_ref[0, :N, :] + p_ref[1, :N, :]) * rin_ref[...]
    mean = agg2[:, :H2] + b2_ref[...]
    log_std = agg2[:, H2:] + b3_ref[...]
    z_ref[...] = mean + noise_ref[...] * jnp.exp(log_std)


def _tc_c(agg2_parts, r_in, b2, b3, noise):
    return pl.pallas_call(
        _tc_c_body,
        out_shape=jax.ShapeDtypeStruct((N, H2), jnp.float32),
    )(agg2_parts, r_in, b2, b3, noise)


BM = 1024
BN = 4096


def _decoder_body(zr_ref, zc_ref, o_ref):
    acc = lax.dot_general(zr_ref[...], zc_ref[...], (((1,), (1,)), ((), ())),
                          preferred_element_type=jnp.float32)
    o_ref[...] = 0.5 * (jnp.tanh(acc * 0.5) + 1.0)


def _decoder(z):
    grid = (pl.cdiv(N, BM), pl.cdiv(N, BN))
    return pl.pallas_call(
        _decoder_body,
        grid=grid,
        in_specs=[
            pl.BlockSpec((BM, H2), lambda i, j: (i, 0)),
            pl.BlockSpec((BN, H2), lambda i, j: (j, 0)),
        ],
        out_specs=pl.BlockSpec((BM, BN), lambda i, j: (i, j)),
        out_shape=jax.ShapeDtypeStruct((N, N), jnp.float32),
    )(z, z)


def kernel(features, edge_index, W1, b1, W2, b2, W3, b3):
    src = edge_index[0]
    dst = edge_index[1]
    W23 = jnp.concatenate([W2, W3], axis=1)
    b1r = jnp.reshape(b1, (1, H1))
    b2r = jnp.reshape(b2, (1, H2))
    b3r = jnp.reshape(b3, (1, H2))
    noise = jax.random.normal(jax.random.key(42), (N, H2), dtype=jnp.float32)

    deg_parts = _sc_degrees(src, dst)
    h1pre, r_out, r_in = _tc_a(deg_parts, features, W1)
    agg1_parts = _sc_agg(h1pre, src, dst)
    m = _tc_b(agg1_parts, r_in, r_out, b1r, W23)
    agg2_parts = _sc_agg(m, src, dst)
    z = _tc_c(agg2_parts, r_in, b2r, b3r, noise)
    return _decoder(z)


# speedup vs baseline: 1.0621x; 1.0621x over previous
import functools

import jax
import jax.numpy as jnp
from jax import lax
from jax.experimental import pallas as pl
from jax.experimental.pallas import tpu as pltpu
from jax.experimental.pallas import tpu_sc as plsc

N = 10000
E = 320000
IN_DIM, H1, H2 = 128, 64, 32

NC, NS, LANES = 2, 16, 16
NW = NC * NS
NPAD = 10240
EPW = E // NW
CH = 128
NFULL = EPW // CH
TAIL = EPW - NFULL * CH

@functools.lru_cache(maxsize=None)
def _sc_mesh():
    return plsc.VectorSubcoreMesh(core_axis_name="c", subcore_axis_name="s",
                                  num_cores=NC, num_subcores=NS)


def _deg_body(ei_hbm, out_hbm, sidx_v, didx_v, hs_v, hd_v,
              isem, isem2):
    c = lax.axis_index("c")
    s = lax.axis_index("s")
    wid = s * NC + c
    zeros = jnp.zeros((LANES,), jnp.float32)
    ones = jnp.ones((LANES,), jnp.float32)
    sdesc = pltpu.async_copy(ei_hbm.at[0, pl.ds(wid * EPW, EPW)], sidx_v, isem)
    ddesc = pltpu.async_copy(ei_hbm.at[1, pl.ds(wid * EPW, EPW)], didx_v,
                             isem2)

    def zero_body(i):
        hs_v[pl.ds(i * LANES, LANES)] = zeros
        hd_v[pl.ds(i * LANES, LANES)] = zeros
    pl.loop(0, NPAD // LANES, unroll=8)(zero_body)

    sdesc.wait()

    def scat_s(i):
        idx = sidx_v[pl.ds(i * LANES, LANES)]
        plsc.addupdate_scatter(hs_v, [idx], ones)
    pl.loop(0, EPW // LANES, unroll=8)(scat_s)
    pltpu.sync_copy(hs_v, out_hbm.at[0, wid])

    ddesc.wait()

    def scat_d(i):
        idx = didx_v[pl.ds(i * LANES, LANES)]
        plsc.addupdate_scatter(hd_v, [idx], ones)
    pl.loop(0, EPW // LANES, unroll=8)(scat_d)
    pltpu.sync_copy(hd_v, out_hbm.at[1, wid])


@functools.lru_cache(maxsize=None)
def _sc_degrees_kernel():
    return pl.kernel(
        _deg_body,
        out_type=jax.ShapeDtypeStruct((2, NW, NPAD), jnp.float32),
        mesh=_sc_mesh(),
        compiler_params=pltpu.CompilerParams(needs_layout_passes=False,
                                             use_tc_tiling_on_sc=False),
        scratch_types=[
            pltpu.VMEM((EPW,), jnp.int32),
            pltpu.VMEM((EPW,), jnp.int32),
            pltpu.VMEM((NPAD,), jnp.float32),
            pltpu.VMEM((NPAD,), jnp.float32),
            pltpu.SemaphoreType.DMA,
            pltpu.SemaphoreType.DMA,
        ],
    )


def _sc_degrees(edge_index):
    return _sc_degrees_kernel()(edge_index)


ROWS_PER_SUB = NPAD // NS


ACH = 200
ANCH = EPW // ACH
DEPTH = 5
ZB_ROWS = 64


def _agg_body(msg_hbm, ei_hbm, out_hbm,
              sidx_v, didx_v, rows0, rows1, rows2, rows3, rows4, zb_v, acc_sh,
              gsem, ssem):
    c = lax.axis_index("c")
    s = lax.axis_index("s")
    wid = s * NC + c
    zeros = jnp.zeros((LANES,), jnp.float32)
    rows = (rows0, rows1, rows2, rows3, rows4)

    pltpu.sync_copy(ei_hbm.at[0, pl.ds(wid * EPW, EPW)], sidx_v)
    pltpu.sync_copy(ei_hbm.at[1, pl.ds(wid * EPW, EPW)], didx_v)

    def issue(k):
        return pltpu.async_copy(msg_hbm.at[sidx_v.at[pl.ds(k * ACH, ACH)]],
                                rows[k % DEPTH], gsem)

    gdescs = [None] * DEPTH
    sdescs = [None] * DEPTH
    for k in range(DEPTH - 1):
        gdescs[k] = issue(k)

    def zero_body(i):
        for j in range(H1 // LANES):
            zb_v[i, pl.ds(j * LANES, LANES)] = zeros
    pl.loop(0, ZB_ROWS)(zero_body)
    for q in range(ROWS_PER_SUB // ZB_ROWS):
        pltpu.sync_copy(zb_v, acc_sh.at[pl.ds(s * ROWS_PER_SUB + q * ZB_ROWS,
                                              ZB_ROWS)])
    plsc.subcore_barrier()

    for k in range(ANCH):
        b = k % DEPTH
        ka = k + DEPTH - 1
        if ka < ANCH:
            ba = ka % DEPTH
            if sdescs[ba] is not None:
                sdescs[ba].wait()
            gdescs[ba] = issue(ka)
        gdescs[b].wait()
        sdescs[b] = pltpu.async_copy(
            rows[b], acc_sh.at[didx_v.at[pl.ds(k * ACH, ACH)]], ssem,
            add=True)
    for k in range(max(0, ANCH - DEPTH), ANCH):
        sdescs[k % DEPTH].wait()
    plsc.subcore_barrier()
    pltpu.sync_copy(acc_sh.at[pl.ds(s * ROWS_PER_SUB, ROWS_PER_SUB)],
                    out_hbm.at[c, pl.ds(s * ROWS_PER_SUB, ROWS_PER_SUB)])


@functools.lru_cache(maxsize=None)
def _sc_agg_kernel():
    return pl.kernel(
        _agg_body,
        out_type=jax.ShapeDtypeStruct((NC, NPAD, H1), jnp.float32),
        mesh=_sc_mesh(),
        compiler_params=pltpu.CompilerParams(needs_layout_passes=False,
                                             use_tc_tiling_on_sc=False),
        scratch_types=[
            pltpu.VMEM((EPW,), jnp.int32),
            pltpu.VMEM((EPW,), jnp.int32),
            pltpu.VMEM((ACH, H1), jnp.float32),
            pltpu.VMEM((ACH, H1), jnp.float32),
            pltpu.VMEM((ACH, H1), jnp.float32),
            pltpu.VMEM((ACH, H1), jnp.float32),
            pltpu.VMEM((ACH, H1), jnp.float32),
            pltpu.VMEM((ZB_ROWS, H1), jnp.float32),
            pltpu.VMEM_SHARED((NPAD, H1), jnp.float32),
            pltpu.SemaphoreType.DMA,
            pltpu.SemaphoreType.DMA,
        ],
    )


def _sc_agg(msg, edge_index):
    return _sc_agg_kernel()(msg, edge_index)


def _tc_a_body(deg_ref, x_ref, w1_ref, h1pre_ref, rout_ref, rin_ref):
    deg = jnp.sum(deg_ref[...], axis=1)
    r = lax.rsqrt(jnp.maximum(deg, 1.0))
    r_out = jnp.reshape(r[0, :N], (N, 1))
    r_in = jnp.reshape(r[1, :N], (N, 1))
    rout_ref[...] = r_out
    rin_ref[...] = r_in
    h1pre_ref[...] = jnp.dot(x_ref[...] * r_out, w1_ref[...],
                             preferred_element_type=jnp.float32)


def _tc_a(deg_parts, features, W1):
    return pl.pallas_call(
        _tc_a_body,
        out_shape=(
            jax.ShapeDtypeStruct((N, H1), jnp.float32),
            jax.ShapeDtypeStruct((N, 1), jnp.float32),
            jax.ShapeDtypeStruct((N, 1), jnp.float32),
        ),
    )(deg_parts, features, W1)


def _tc_b_body(p_ref, rin_ref, rout_ref, b1_ref, w23_ref, m_ref):
    agg1 = p_ref[0, :N, :] + p_ref[1, :N, :]
    h = jnp.maximum(agg1 * rin_ref[...] + b1_ref[...], 0.0)
    m_ref[...] = jnp.dot(h * rout_ref[...], w23_ref[...],
                         preferred_element_type=jnp.float32)


def _tc_b(agg1_parts, r_in, r_out, b1, W23):
    return pl.pallas_call(
        _tc_b_body,
        out_shape=jax.ShapeDtypeStruct((N, H1), jnp.float32),
    )(agg1_parts, r_in, r_out, b1, W23)


def _tc_c_body(p_ref, rin_ref, b2_ref, b3_ref, noise_ref, z_ref):
    agg2 = (p_ref[0, :N, :] + p_ref[1, :N, :]) * rin_ref[...]
    mean = agg2[:, :H2] + b2_ref[...]
    log_std = agg2[:, H2:] + b3_ref[...]
    z_ref[...] = mean + noise_ref[...] * jnp.exp(log_std)


def _tc_c(agg2_parts, r_in, b2, b3, noise):
    return pl.pallas_call(
        _tc_c_body,
        out_shape=jax.ShapeDtypeStruct((N, H2), jnp.float32),
    )(agg2_parts, r_in, b2, b3, noise)


BM = 2048
BN = 2048


def _decoder_body(zr_ref, zc_ref, o_ref):
    acc = lax.dot_general(zr_ref[...], zc_ref[...], (((1,), (1,)), ((), ())),
                          preferred_element_type=jnp.float32)
    o_ref[...] = 0.5 * (jnp.tanh(acc * 0.5) + 1.0)


def _decoder(z):
    grid = (pl.cdiv(N, BM), pl.cdiv(N, BN))
    return pl.pallas_call(
        _decoder_body,
        grid=grid,
        in_specs=[
            pl.BlockSpec((BM, H2), lambda i, j: (i, 0)),
            pl.BlockSpec((BN, H2), lambda i, j: (j, 0)),
        ],
        out_specs=pl.BlockSpec((BM, BN), lambda i, j: (i, j)),
        out_shape=jax.ShapeDtypeStruct((N, N), jnp.float32),
    )(z, z)


def kernel(features, edge_index, W1, b1, W2, b2, W3, b3):
    W23 = jnp.concatenate([W2, W3], axis=1)
    b1r = jnp.reshape(b1, (1, H1))
    b2r = jnp.reshape(b2, (1, H2))
    b3r = jnp.reshape(b3, (1, H2))
    noise = jax.random.normal(jax.random.key(42), (N, H2), dtype=jnp.float32)

    deg_parts = _sc_degrees(edge_index)
    h1pre, r_out, r_in = _tc_a(deg_parts, features, W1)
    agg1_parts = _sc_agg(h1pre, edge_index)
    m = _tc_b(agg1_parts, r_in, r_out, b1r, W23)
    agg2_parts = _sc_agg(m, edge_index)
    z = _tc_c(agg2_parts, r_in, b2r, b3r, noise)
    return _decoder(z)


# speedup vs baseline: 1.0635x; 1.0013x over previous
import functools

import jax
import jax.numpy as jnp
from jax import lax
from jax.experimental import pallas as pl
from jax.experimental.pallas import tpu as pltpu
from jax.experimental.pallas import tpu_sc as plsc

N = 10000
E = 320000
IN_DIM, H1, H2 = 128, 64, 32

NC, NS, LANES = 2, 16, 16
NW = NC * NS
NPAD = 10240
EPW = E // NW

@functools.lru_cache(maxsize=None)
def _sc_mesh():
    return plsc.VectorSubcoreMesh(core_axis_name="c", subcore_axis_name="s",
                                  num_cores=NC, num_subcores=NS)


def _deg_body(ei_hbm, out_hbm, sidx_v, didx_v, hs_v, hd_v,
              isem, isem2):
    c = lax.axis_index("c")
    s = lax.axis_index("s")
    wid = s * NC + c
    zeros = jnp.zeros((LANES,), jnp.float32)
    ones = jnp.ones((LANES,), jnp.float32)
    sdesc = pltpu.async_copy(ei_hbm.at[0, pl.ds(wid * EPW, EPW)], sidx_v, isem)
    ddesc = pltpu.async_copy(ei_hbm.at[1, pl.ds(wid * EPW, EPW)], didx_v,
                             isem2)

    def zero_body(i):
        hs_v[pl.ds(i * LANES, LANES)] = zeros
        hd_v[pl.ds(i * LANES, LANES)] = zeros
    pl.loop(0, NPAD // LANES, unroll=8)(zero_body)

    sdesc.wait()

    def scat_s(i):
        idx = sidx_v[pl.ds(i * LANES, LANES)]
        plsc.addupdate_scatter(hs_v, [idx], ones)
    pl.loop(0, EPW // LANES, unroll=8)(scat_s)
    pltpu.sync_copy(hs_v, out_hbm.at[0, wid])

    ddesc.wait()

    def scat_d(i):
        idx = didx_v[pl.ds(i * LANES, LANES)]
        plsc.addupdate_scatter(hd_v, [idx], ones)
    pl.loop(0, EPW // LANES, unroll=8)(scat_d)
    pltpu.sync_copy(hd_v, out_hbm.at[1, wid])


@functools.lru_cache(maxsize=None)
def _sc_degrees_kernel():
    return pl.kernel(
        _deg_body,
        out_type=jax.ShapeDtypeStruct((2, NW, NPAD), jnp.float32),
        mesh=_sc_mesh(),
        compiler_params=pltpu.CompilerParams(needs_layout_passes=False,
                                             use_tc_tiling_on_sc=False),
        scratch_types=[
            pltpu.VMEM((EPW,), jnp.int32),
            pltpu.VMEM((EPW,), jnp.int32),
            pltpu.VMEM((NPAD,), jnp.float32),
            pltpu.VMEM((NPAD,), jnp.float32),
            pltpu.SemaphoreType.DMA,
            pltpu.SemaphoreType.DMA,
        ],
    )


def _sc_degrees(edge_index):
    return _sc_degrees_kernel()(edge_index)


ROWS_PER_SUB = NPAD // NS


ACH = 200
ANCH = EPW // ACH
DEPTH = 5
ZB_ROWS = 64


def _agg_body(msg_hbm, ei_hbm, out_hbm,
              sidx_v, didx_v, rows0, rows1, rows2, rows3, rows4, zb_v, acc_sh,
              gsem, ssem):
    c = lax.axis_index("c")
    s = lax.axis_index("s")
    wid = s * NC + c
    zeros = jnp.zeros((LANES,), jnp.float32)
    rows = (rows0, rows1, rows2, rows3, rows4)

    pltpu.sync_copy(ei_hbm.at[0, pl.ds(wid * EPW, EPW)], sidx_v)
    pltpu.sync_copy(ei_hbm.at[1, pl.ds(wid * EPW, EPW)], didx_v)

    def issue(k):
        return pltpu.async_copy(msg_hbm.at[sidx_v.at[pl.ds(k * ACH, ACH)]],
                                rows[k % DEPTH], gsem)

    gdescs = [None] * DEPTH
    sdescs = [None] * DEPTH
    for k in range(DEPTH - 1):
        gdescs[k] = issue(k)

    def zero_body(i):
        for j in range(H1 // LANES):
            zb_v[i, pl.ds(j * LANES, LANES)] = zeros
    pl.loop(0, ZB_ROWS)(zero_body)
    for q in range(ROWS_PER_SUB // ZB_ROWS):
        pltpu.sync_copy(zb_v, acc_sh.at[pl.ds(s * ROWS_PER_SUB + q * ZB_ROWS,
                                              ZB_ROWS)])
    plsc.subcore_barrier()

    for k in range(ANCH):
        b = k % DEPTH
        ka = k + DEPTH - 1
        if ka < ANCH:
            ba = ka % DEPTH
            if sdescs[ba] is not None:
                sdescs[ba].wait()
            gdescs[ba] = issue(ka)
        gdescs[b].wait()
        sdescs[b] = pltpu.async_copy(
            rows[b], acc_sh.at[didx_v.at[pl.ds(k * ACH, ACH)]], ssem,
            add=True)
    for k in range(max(0, ANCH - DEPTH), ANCH):
        sdescs[k % DEPTH].wait()
    plsc.subcore_barrier()
    pltpu.sync_copy(acc_sh.at[pl.ds(s * ROWS_PER_SUB, ROWS_PER_SUB)],
                    out_hbm.at[c, pl.ds(s * ROWS_PER_SUB, ROWS_PER_SUB)])


@functools.lru_cache(maxsize=None)
def _sc_agg_kernel():
    return pl.kernel(
        _agg_body,
        out_type=jax.ShapeDtypeStruct((NC, NPAD, H1), jnp.float32),
        mesh=_sc_mesh(),
        compiler_params=pltpu.CompilerParams(needs_layout_passes=False,
                                             use_tc_tiling_on_sc=False),
        scratch_types=[
            pltpu.VMEM((EPW,), jnp.int32),
            pltpu.VMEM((EPW,), jnp.int32),
            pltpu.VMEM((ACH, H1), jnp.float32),
            pltpu.VMEM((ACH, H1), jnp.float32),
            pltpu.VMEM((ACH, H1), jnp.float32),
            pltpu.VMEM((ACH, H1), jnp.float32),
            pltpu.VMEM((ACH, H1), jnp.float32),
            pltpu.VMEM((ZB_ROWS, H1), jnp.float32),
            pltpu.VMEM_SHARED((NPAD, H1), jnp.float32),
            pltpu.SemaphoreType.DMA,
            pltpu.SemaphoreType.DMA,
        ],
    )


def _sc_agg(msg, edge_index):
    return _sc_agg_kernel()(msg, edge_index)


def _tc_a_body(deg_ref, x_ref, w1_ref, h1pre_ref, rout_ref, rin_ref):
    deg = jnp.sum(deg_ref[...], axis=1)
    r = lax.rsqrt(jnp.maximum(deg, 1.0))
    r_out = jnp.reshape(r[0, :N], (N, 1))
    r_in = jnp.reshape(r[1, :N], (N, 1))
    rout_ref[...] = r_out
    rin_ref[...] = r_in
    h1pre_ref[...] = jnp.dot(x_ref[...] * r_out, w1_ref[...],
                             preferred_element_type=jnp.float32)


def _tc_a(deg_parts, features, W1):
    return pl.pallas_call(
        _tc_a_body,
        out_shape=(
            jax.ShapeDtypeStruct((N, H1), jnp.float32),
            jax.ShapeDtypeStruct((N, 1), jnp.float32),
            jax.ShapeDtypeStruct((N, 1), jnp.float32),
        ),
    )(deg_parts, features, W1)


def _tc_b_body(p_ref, rin_ref, rout_ref, b1_ref, w23_ref, m_ref):
    agg1 = p_ref[0, :N, :] + p_ref[1, :N, :]
    h = jnp.maximum(agg1 * rin_ref[...] + b1_ref[...], 0.0)
    m_ref[...] = jnp.dot(h * rout_ref[...], w23_ref[...],
                         preferred_element_type=jnp.float32)


def _tc_b(agg1_parts, r_in, r_out, b1, W23):
    return pl.pallas_call(
        _tc_b_body,
        out_shape=jax.ShapeDtypeStruct((N, H1), jnp.float32),
    )(agg1_parts, r_in, r_out, b1, W23)


def _tc_c_body(p_ref, rin_ref, b2_ref, b3_ref, noise_ref, z_ref):
    agg2 = (p_ref[0, :N, :] + p_ref[1, :N, :]) * rin_ref[...]
    mean = agg2[:, :H2] + b2_ref[...]
    log_std = agg2[:, H2:] + b3_ref[...]
    z_ref[...] = mean + noise_ref[...] * jnp.exp(log_std)


def _tc_c(agg2_parts, r_in, b2, b3, noise):
    return pl.pallas_call(
        _tc_c_body,
        out_shape=jax.ShapeDtypeStruct((N, H2), jnp.float32),
    )(agg2_parts, r_in, b2, b3, noise)


BM = 2048
BN = 2048


def _decoder_body(zr_ref, zc_ref, o_ref):
    acc = lax.dot_general(zr_ref[...], zc_ref[...], (((1,), (1,)), ((), ())),
                          preferred_element_type=jnp.float32)
    o_ref[...] = 0.5 * (jnp.tanh(acc * 0.5) + 1.0)


def _decoder(z):
    grid = (pl.cdiv(N, BM), pl.cdiv(N, BN))
    return pl.pallas_call(
        _decoder_body,
        grid=grid,
        in_specs=[
            pl.BlockSpec((BM, H2), lambda i, j: (i, 0)),
            pl.BlockSpec((BN, H2), lambda i, j: (j, 0)),
        ],
        out_specs=pl.BlockSpec((BM, BN), lambda i, j: (i, j)),
        out_shape=jax.ShapeDtypeStruct((N, N), jnp.float32),
    )(z, z)


def kernel(features, edge_index, W1, b1, W2, b2, W3, b3):
    W23 = jnp.concatenate([W2, W3], axis=1)
    b1r = jnp.reshape(b1, (1, H1))
    b2r = jnp.reshape(b2, (1, H2))
    b3r = jnp.reshape(b3, (1, H2))
    noise = jax.random.normal(jax.random.key(42), (N, H2), dtype=jnp.float32)

    deg_parts = _sc_degrees(edge_index)
    h1pre, r_out, r_in = _tc_a(deg_parts, features, W1)
    agg1_parts = _sc_agg(h1pre, edge_index)
    m = _tc_b(agg1_parts, r_in, r_out, b1r, W23)
    agg2_parts = _sc_agg(m, edge_index)
    z = _tc_c(agg2_parts, r_in, b2r, b3r, noise)
    return _decoder(z)
